# Initial kernel scaffold; baseline (speedup 1.0000x reference)
import jax
import jax.numpy as jnp
from jax.experimental import pallas as pl


def kernel(token_idx, embedding, fc_w, fc_b):
    raise NotImplementedError("write your pallas kernel here")


# speedup vs baseline: 9.1874x; 9.1874x over previous
import functools

import jax
import jax.numpy as jnp
from jax import lax
from jax.experimental import pallas as pl
from jax.experimental.pallas import tpu as pltpu
from jax.experimental.pallas import tpu_sc as plsc

_VOCAB = 1_000_000
_D = 16
_L = 200
_B = 16384
_NCLS = 3

_NC = 2
_NS = 16
_NW = _NC * _NS

_ROWS_PER_W = _B // _NW
_TOK_PER_W = _ROWS_PER_W * _L
_CHUNK_ROWS = 8
_CHUNK_TOK = _CHUNK_ROWS * _L
_NCHUNK = _ROWS_PER_W // _CHUNK_ROWS

_INV_L = 1.0 / _L


def _body(tok_hbm, table_hbm, wt_hbm, out_hbm,
          idx0, idx1, rows0, rows1, out_v, wt_v, sem0, sem1):
  cid = lax.axis_index("c")
  sid = lax.axis_index("s")
  wid = sid * _NC + cid
  tok_base = wid * _TOK_PER_W
  row_base = wid * _ROWS_PER_W

  pltpu.sync_copy(wt_hbm, wt_v)
  w0 = wt_v[0]
  w1 = wt_v[1]
  w2 = wt_v[2]
  bias = wt_v[3]
  lane = lax.iota(jnp.int32, 16)
  perms = tuple(lane ^ sh for sh in (8, 4, 2, 1))

  dnums = lax.GatherDimensionNumbers(
      offset_dims=(), collapsed_slice_dims=(0,), start_index_map=(0,))

  def shuffle(x, p):
    return lax.gather(x, p[:, None], dnums, slice_sizes=(1,),
                      unique_indices=True, indices_are_sorted=False,
                      mode=lax.GatherScatterMode.PROMISE_IN_BOUNDS)

  def allsum(x):
    for p in perms:
      x = x + shuffle(x, p)
    return x

  idx = (idx0, idx1)
  rows = (rows0, rows1)
  sems = (sem0, sem1)

  def gather_start(p):
    pltpu.make_async_copy(table_hbm.at[idx[p]], rows[p], sems[p]).start()

  def gather_wait(p):
    pltpu.make_async_copy(table_hbm.at[idx[p]], rows[p], sems[p]).wait()

  def idx_load(c, p):
    pltpu.sync_copy(tok_hbm.at[pl.ds(tok_base + c * _CHUNK_TOK, _CHUNK_TOK)],
                    idx[p])

  def accumulate(c, p):
    rows_ref = rows[p]
    for r in range(_CHUNK_ROWS):
      base_row = r * _L

      def inner(i, accs):
        a0, a1, a2, a3 = accs
        o = base_row + i * 4
        return (a0 + rows_ref[o], a1 + rows_ref[o + 1],
                a2 + rows_ref[o + 2], a3 + rows_ref[o + 3])

      z = jnp.zeros((16,), jnp.float32)
      a0, a1, a2, a3 = lax.fori_loop(0, _L // 4, inner, (z, z, z, z))
      x = ((a0 + a1) + (a2 + a3)) * _INV_L
      s0 = allsum(x * w0)
      s1 = allsum(x * w1)
      s2 = allsum(x * w2)
      zero = jnp.zeros((16,), jnp.float32)
      res = (bias + jnp.where(lane == 0, s0, zero)
             + jnp.where(lane == 1, s1, zero)
             + jnp.where(lane == 2, s2, zero))
      out_v[c * _CHUNK_ROWS + r] = res

  idx_load(0, 0)
  gather_start(0)
  idx_load(1, 1)

  def chunk_step(c, p):
    gather_start(1 - p)
    gather_wait(p)
    idx_load(c + 2, p)
    accumulate(c, p)

  def body_k(k, carry):
    c = 2 * k
    chunk_step(c, 0)
    chunk_step(c + 1, 1)
    return carry

  lax.fori_loop(0, (_NCHUNK - 2) // 2, body_k, 0)

  gather_start(1)
  gather_wait(0)
  accumulate(_NCHUNK - 2, 0)
  gather_wait(1)
  accumulate(_NCHUNK - 1, 1)

  pltpu.sync_copy(out_v, out_hbm.at[pl.ds(row_base, _ROWS_PER_W)])


@functools.partial(
    pl.kernel,
    out_type=jax.ShapeDtypeStruct((_B, 16), jnp.float32),
    mesh=plsc.VectorSubcoreMesh(core_axis_name="c", subcore_axis_name="s"),
    scratch_types=[
        pltpu.VMEM((_CHUNK_TOK,), jnp.int32),
        pltpu.VMEM((_CHUNK_TOK,), jnp.int32),
        pltpu.VMEM((_CHUNK_TOK, 16), jnp.float32),
        pltpu.VMEM((_CHUNK_TOK, 16), jnp.float32),
        pltpu.VMEM((_ROWS_PER_W, 16), jnp.float32),
        pltpu.VMEM((4, 16), jnp.float32),
        pltpu.SemaphoreType.DMA,
        pltpu.SemaphoreType.DMA,
    ],
    compiler_params=pltpu.CompilerParams(use_tc_tiling_on_sc=False),
)
def _sc_embed_mean_fc(tok_hbm, table_hbm, wt_hbm, out_hbm, *scratch):
  _body(tok_hbm, table_hbm, wt_hbm, out_hbm, *scratch)


@jax.jit
def kernel(token_idx, embedding, fc_w, fc_b):
  tok = token_idx.reshape(-1).astype(jnp.int32)
  wt = jnp.zeros((4, 16), jnp.float32)
  wt = wt.at[:_NCLS, :].set(fc_w.T.astype(jnp.float32))
  wt = wt.at[3, :_NCLS].set(fc_b.astype(jnp.float32))
  out16 = _sc_embed_mean_fc(tok, embedding, wt)
  return out16[:, :_NCLS]


# speedup vs baseline: 9.3121x; 1.0136x over previous
import functools

import jax
import jax.numpy as jnp
from jax import lax
from jax.experimental import pallas as pl
from jax.experimental.pallas import tpu as pltpu
from jax.experimental.pallas import tpu_sc as plsc

_VOCAB = 1_000_000
_D = 16
_L = 200
_B = 16384
_NCLS = 3

_NC = 2
_NS = 16
_NW = _NC * _NS

_ROWS_PER_W = _B // _NW
_TOK_PER_W = _ROWS_PER_W * _L
_CHUNK_ROWS = 8
_CHUNK_TOK = _CHUNK_ROWS * _L
_NCHUNK = _ROWS_PER_W // _CHUNK_ROWS

_INV_L = 1.0 / _L


def _body(tok_hbm, table_hbm, wt_hbm, out_hbm,
          idx0, idx1, rows0, rows1, out_v, wt_v, sem0, sem1):
  cid = lax.axis_index("c")
  sid = lax.axis_index("s")
  wid = sid * _NC + cid
  tok_base = wid * _TOK_PER_W
  row_base = wid * _ROWS_PER_W

  pltpu.sync_copy(wt_hbm, wt_v)
  w0 = wt_v[0]
  w1 = wt_v[1]
  w2 = wt_v[2]
  bias = wt_v[3]
  lane = lax.iota(jnp.int32, 16)
  perms = tuple(lane ^ sh for sh in (8, 4, 2, 1))

  dnums = lax.GatherDimensionNumbers(
      offset_dims=(), collapsed_slice_dims=(0,), start_index_map=(0,))

  def shuffle(x, p):
    return lax.gather(x, p[:, None], dnums, slice_sizes=(1,),
                      unique_indices=True, indices_are_sorted=False,
                      mode=lax.GatherScatterMode.PROMISE_IN_BOUNDS)

  def allsum(x):
    for p in perms:
      x = x + shuffle(x, p)
    return x

  idx = (idx0, idx1)
  rows = (rows0, rows1)
  sems = (sem0, sem1)

  def gather_start(p):
    pltpu.make_async_copy(table_hbm.at[idx[p]], rows[p], sems[p]).start()

  def gather_wait(p):
    pltpu.make_async_copy(table_hbm.at[idx[p]], rows[p], sems[p]).wait()

  def idx_load(c, p):
    pltpu.sync_copy(tok_hbm.at[pl.ds(tok_base + c * _CHUNK_TOK, _CHUNK_TOK)],
                    idx[p])

  def accumulate(c, p):
    rows_ref = rows[p]
    for r in range(_CHUNK_ROWS):
      base_row = r * _L

      def inner(i, accs):
        o = base_row + i * 8
        return tuple(a + rows_ref[o + j] for j, a in enumerate(accs))

      z = jnp.zeros((16,), jnp.float32)
      accs = lax.fori_loop(0, _L // 8, inner, (z,) * 8)
      a = (((accs[0] + accs[1]) + (accs[2] + accs[3]))
           + ((accs[4] + accs[5]) + (accs[6] + accs[7])))
      x = a * _INV_L
      s0 = allsum(x * w0)
      s1 = allsum(x * w1)
      s2 = allsum(x * w2)
      zero = jnp.zeros((16,), jnp.float32)
      res = (bias + jnp.where(lane == 0, s0, zero)
             + jnp.where(lane == 1, s1, zero)
             + jnp.where(lane == 2, s2, zero))
      out_v[c * _CHUNK_ROWS + r] = res

  idx_load(0, 0)
  gather_start(0)
  idx_load(1, 1)

  def chunk_step(c, p):
    gather_start(1 - p)
    gather_wait(p)
    idx_load(c + 2, p)
    accumulate(c, p)

  def body_k(k, carry):
    c = 2 * k
    chunk_step(c, 0)
    chunk_step(c + 1, 1)
    return carry

  lax.fori_loop(0, (_NCHUNK - 2) // 2, body_k, 0)

  gather_start(1)
  gather_wait(0)
  accumulate(_NCHUNK - 2, 0)
  gather_wait(1)
  accumulate(_NCHUNK - 1, 1)

  pltpu.sync_copy(out_v, out_hbm.at[pl.ds(row_base, _ROWS_PER_W)])


@functools.partial(
    pl.kernel,
    out_type=jax.ShapeDtypeStruct((_B, 16), jnp.float32),
    mesh=plsc.VectorSubcoreMesh(core_axis_name="c", subcore_axis_name="s"),
    scratch_types=[
        pltpu.VMEM((_CHUNK_TOK,), jnp.int32),
        pltpu.VMEM((_CHUNK_TOK,), jnp.int32),
        pltpu.VMEM((_CHUNK_TOK, 16), jnp.float32),
        pltpu.VMEM((_CHUNK_TOK, 16), jnp.float32),
        pltpu.VMEM((_ROWS_PER_W, 16), jnp.float32),
        pltpu.VMEM((4, 16), jnp.float32),
        pltpu.SemaphoreType.DMA,
        pltpu.SemaphoreType.DMA,
    ],
    compiler_params=pltpu.CompilerParams(use_tc_tiling_on_sc=False),
)
def _sc_embed_mean_fc(tok_hbm, table_hbm, wt_hbm, out_hbm, *scratch):
  _body(tok_hbm, table_hbm, wt_hbm, out_hbm, *scratch)


@jax.jit
def kernel(token_idx, embedding, fc_w, fc_b):
  tok = token_idx.reshape(-1).astype(jnp.int32)
  wt = jnp.zeros((4, 16), jnp.float32)
  wt = wt.at[:_NCLS, :].set(fc_w.T.astype(jnp.float32))
  wt = wt.at[3, :_NCLS].set(fc_b.astype(jnp.float32))
  out16 = _sc_embed_mean_fc(tok, embedding, wt)
  return out16[:, :_NCLS]


# speedup vs baseline: 9.6294x; 1.0341x over previous
import functools

import jax
import jax.numpy as jnp
from jax import lax
from jax.experimental import pallas as pl
from jax.experimental.pallas import tpu as pltpu
from jax.experimental.pallas import tpu_sc as plsc

_VOCAB = 1_000_000
_D = 16
_L = 200
_B = 16384
_NCLS = 3

_NC = 2
_NS = 16
_NW = _NC * _NS

_ROWS_PER_W = _B // _NW
_TOK_PER_W = _ROWS_PER_W * _L
_CHUNK_ROWS = 8
_CHUNK_TOK = _CHUNK_ROWS * _L
_NCHUNK = _ROWS_PER_W // _CHUNK_ROWS

_INV_L = 1.0 / _L


_NBUF = 4


def _body(tok_hbm, table_hbm, wt_hbm, out_hbm, *refs):
  idx = refs[0:_NBUF]
  rows = refs[_NBUF:2 * _NBUF]
  out_v = refs[2 * _NBUF]
  wt_v = refs[2 * _NBUF + 1]
  sems = refs[2 * _NBUF + 2:]
  cid = lax.axis_index("c")
  sid = lax.axis_index("s")
  wid = sid * _NC + cid
  tok_base = wid * _TOK_PER_W
  row_base = wid * _ROWS_PER_W

  pltpu.sync_copy(wt_hbm, wt_v)
  w0 = wt_v[0]
  w1 = wt_v[1]
  w2 = wt_v[2]
  bias = wt_v[3]
  lane = lax.iota(jnp.int32, 16)
  perms = tuple(lane ^ sh for sh in (8, 4, 2, 1))

  dnums = lax.GatherDimensionNumbers(
      offset_dims=(), collapsed_slice_dims=(0,), start_index_map=(0,))

  def shuffle(x, p):
    return lax.gather(x, p[:, None], dnums, slice_sizes=(1,),
                      unique_indices=True, indices_are_sorted=False,
                      mode=lax.GatherScatterMode.PROMISE_IN_BOUNDS)

  def allsum(x):
    for p in perms:
      x = x + shuffle(x, p)
    return x

  def gather_start(p):
    pltpu.make_async_copy(table_hbm.at[idx[p]], rows[p], sems[p]).start()

  def gather_wait(p):
    pltpu.make_async_copy(table_hbm.at[idx[p]], rows[p], sems[p]).wait()

  def idx_load(c, p):
    pltpu.sync_copy(tok_hbm.at[pl.ds(tok_base + c * _CHUNK_TOK, _CHUNK_TOK)],
                    idx[p])

  def accumulate(c, p):
    rows_ref = rows[p]
    for r in range(_CHUNK_ROWS):
      base_row = r * _L

      def inner(i, accs):
        o = base_row + i * 8
        return tuple(a + rows_ref[o + j] for j, a in enumerate(accs))

      z = jnp.zeros((16,), jnp.float32)
      accs = lax.fori_loop(0, _L // 8, inner, (z,) * 8)
      a = (((accs[0] + accs[1]) + (accs[2] + accs[3]))
           + ((accs[4] + accs[5]) + (accs[6] + accs[7])))
      x = a * _INV_L
      s0 = allsum(x * w0)
      s1 = allsum(x * w1)
      s2 = allsum(x * w2)
      zero = jnp.zeros((16,), jnp.float32)
      res = (bias + jnp.where(lane == 0, s0, zero)
             + jnp.where(lane == 1, s1, zero)
             + jnp.where(lane == 2, s2, zero))
      out_v[c * _CHUNK_ROWS + r] = res

  for b in range(_NBUF):
    idx_load(b, b)
    gather_start(b)

  def chunk_step(c, p):
    gather_wait(p)
    accumulate(c, p)

    @pl.when(c + _NBUF < _NCHUNK)
    def _():
      idx_load(c + _NBUF, p)
      gather_start(p)

  def body_k(k, carry):
    c0 = _NBUF * k
    for p in range(_NBUF):
      chunk_step(c0 + p, p)
    return carry

  lax.fori_loop(0, _NCHUNK // _NBUF, body_k, 0)

  pltpu.sync_copy(out_v, out_hbm.at[pl.ds(row_base, _ROWS_PER_W)])


@functools.partial(
    pl.kernel,
    out_type=jax.ShapeDtypeStruct((_B, 16), jnp.float32),
    mesh=plsc.VectorSubcoreMesh(core_axis_name="c", subcore_axis_name="s"),
    scratch_types=(
        [pltpu.VMEM((_CHUNK_TOK,), jnp.int32) for _ in range(_NBUF)]
        + [pltpu.VMEM((_CHUNK_TOK, 16), jnp.float32) for _ in range(_NBUF)]
        + [pltpu.VMEM((_ROWS_PER_W, 16), jnp.float32),
           pltpu.VMEM((4, 16), jnp.float32)]
        + [pltpu.SemaphoreType.DMA for _ in range(_NBUF)]
    ),
    compiler_params=pltpu.CompilerParams(use_tc_tiling_on_sc=False),
)
def _sc_embed_mean_fc(tok_hbm, table_hbm, wt_hbm, out_hbm, *scratch):
  _body(tok_hbm, table_hbm, wt_hbm, out_hbm, *scratch)


@jax.jit
def kernel(token_idx, embedding, fc_w, fc_b):
  tok = token_idx.reshape(-1).astype(jnp.int32)
  wt = jnp.zeros((4, 16), jnp.float32)
  wt = wt.at[:_NCLS, :].set(fc_w.T.astype(jnp.float32))
  wt = wt.at[3, :_NCLS].set(fc_b.astype(jnp.float32))
  out16 = _sc_embed_mean_fc(tok, embedding, wt)
  return out16[:, :_NCLS]
